# baseline (speedup 1.0000x reference)
import functools

import jax
import jax.numpy as jnp
import numpy as np
from jax import lax
from jax.experimental import pallas as pl
from jax.experimental.pallas import tpu as pltpu
from jax.experimental.pallas import tpu_sc as plsc

VOCAB = 100000
DIM = 64
BATCH = 1024
CTX = 200

NC = 2
NS = 16
NW = NC * NS
L = 16

BPW = BATCH // NW
CHUNK_A = 104
CHUNK_B = CTX - CHUNK_A
DV = DIM // L


_PERM = np.empty((DIM,), dtype=np.int32)
for _s in range(2):
  for _i in range(16):
    _PERM[32 * _s + 2 * _i] = 32 * _s + _i
    _PERM[32 * _s + 2 * _i + 1] = 32 * _s + 16 + _i


def _sc_pool(table, idx_flat):
  @functools.partial(
      pl.kernel,
      out_type=jax.ShapeDtypeStruct((BATCH, DIM), jnp.float32),
      mesh=plsc.VectorSubcoreMesh(core_axis_name="c", subcore_axis_name="s"),
      compiler_params=pltpu.CompilerParams(
          use_tc_tiling_on_sc=False, needs_layout_passes=False),
      scratch_types=[
          pltpu.VMEM((BPW * CTX,), jnp.int32),
          pltpu.VMEM((2, CTX, DIM), jnp.bfloat16),
          pltpu.VMEM((BPW, DIM), jnp.float32),
          pltpu.SemaphoreType.DMA,
          pltpu.SemaphoreType.DMA,
      ],
  )
  def sc_kernel(table_hbm, idx_hbm, out_hbm, idx_v, rows_v, pooled_v,
                sem0, sem1):
    wid = lax.axis_index("s") * NC + lax.axis_index("c")
    base = wid * BPW
    sems = (sem0, sem1)
    pltpu.sync_copy(idx_hbm.at[pl.ds(base * CTX, BPW * CTX)], idx_v)

    def issue(b, buf):
      off = b * CTX
      pltpu.async_copy(
          table_hbm.at[idx_v.at[pl.ds(off, CHUNK_A)]],
          rows_v.at[buf, pl.ds(0, CHUNK_A)], sems[buf])
      pltpu.async_copy(
          table_hbm.at[idx_v.at[pl.ds(off + CHUNK_A, CHUNK_B)]],
          rows_v.at[buf, pl.ds(CHUNK_A, CHUNK_B)], sems[buf])

    def drain(b, buf):
      off = b * CTX
      pltpu.make_async_copy(
          table_hbm.at[idx_v.at[pl.ds(off, CHUNK_A)]],
          rows_v.at[buf, pl.ds(0, CHUNK_A)], sems[buf]).wait()
      pltpu.make_async_copy(
          table_hbm.at[idx_v.at[pl.ds(off + CHUNK_A, CHUNK_B)]],
          rows_v.at[buf, pl.ds(CHUNK_A, CHUNK_B)], sems[buf]).wait()

    def reduce_into(b, buf):
      def body_j(j, accs):
        out = list(accs)
        for s in range(2):
          a, bb = plsc.unpack(
              rows_v[buf, j, pl.ds(32 * s, 32)],
              format=plsc.PackFormat.INTERLEAVED)
          out[2 * s] = out[2 * s] + a
          out[2 * s + 1] = out[2 * s + 1] + bb
        return tuple(out)

      accs = lax.fori_loop(
          0, CTX, body_j,
          tuple(jnp.zeros((L,), jnp.float32) for _ in range(DV)),
          unroll=4)
      for k in range(DV):
        pooled_v[b, pl.ds(k * L, L)] = accs[k] * (1.0 / CTX)

    issue(0, 0)

    def body_pair(i, carry):
      b0 = 2 * i
      drain(b0, 0)
      issue(b0 + 1, 1)
      reduce_into(b0, 0)
      drain(b0 + 1, 1)

      @pl.when(b0 + 2 < BPW)
      def _():
        issue(b0 + 2, 0)

      reduce_into(b0 + 1, 1)
      return carry

    lax.fori_loop(0, BPW // 2, body_pair, 0)
    pltpu.sync_copy(pooled_v, out_hbm.at[pl.ds(base, BPW)])

  return sc_kernel(table, idx_flat)


V_TILE = 4096
V_GRID = (VOCAB + V_TILE - 1) // V_TILE


def _mm_body(wt_ref, x_ref, b_ref, o_ref):
  o_ref[...] = lax.dot_general(
      wt_ref[...], x_ref[...],
      dimension_numbers=(((0,), (1,)), ((), ())),
      preferred_element_type=jnp.float32) + b_ref[...]


def _tc_project(pooled, Wt, b2d):
  return pl.pallas_call(
      _mm_body,
      grid=(V_GRID,),
      in_specs=[
          pl.BlockSpec((DIM, V_TILE), lambda i: (0, i)),
          pl.BlockSpec((BATCH, DIM), lambda i: (0, 0)),
          pl.BlockSpec((V_TILE, 1), lambda i: (i, 0)),
      ],
      out_specs=pl.BlockSpec((V_TILE, BATCH), lambda i: (i, 0)),
      out_shape=jax.ShapeDtypeStruct((VOCAB, BATCH), jnp.float32),
  )(Wt, pooled, b2d)


def kernel(emb_table, W, b, inputs):
  table_bf = emb_table[:, _PERM].astype(jnp.bfloat16)
  pooled = _sc_pool(table_bf, inputs.reshape(-1))
  out_t = _tc_project(pooled, W.T, b.reshape(VOCAB, 1))
  return out_t.T


# speedup vs baseline: 1.1218x; 1.1218x over previous
import functools

import jax
import jax.numpy as jnp
from jax import lax
from jax.experimental import pallas as pl
from jax.experimental.pallas import tpu as pltpu
from jax.experimental.pallas import tpu_sc as plsc

VOCAB = 100000
DIM = 64
BATCH = 1024
CTX = 200

NC = 2
NS = 16
NW = NC * NS
L = 16

N_PASS = DIM // NW
BLK = 64
NB = BATCH // BLK
CB = BLK // L


def _sc_pool_t(tableT, idxT):
  @functools.partial(
      pl.kernel,
      out_type=jax.ShapeDtypeStruct((DIM, BATCH), jnp.float32),
      mesh=plsc.VectorSubcoreMesh(core_axis_name="c", subcore_axis_name="s"),
      compiler_params=pltpu.CompilerParams(
          use_tc_tiling_on_sc=False, needs_layout_passes=False),
      scratch_types=[
          pltpu.VMEM((VOCAB,), jnp.float32),
          pltpu.VMEM((2, CTX, BLK), jnp.int32),
          pltpu.VMEM((BATCH,), jnp.float32),
          pltpu.SemaphoreType.DMA,
          pltpu.SemaphoreType.DMA,
      ],
  )
  def sc_kernel(tableT_hbm, idxT_hbm, out_hbm, row_v, idx_v, pooled_v,
                sem0, sem1):
    wid = lax.axis_index("s") * NC + lax.axis_index("c")
    sems = (sem0, sem1)

    def issue(g, buf):
      pltpu.async_copy(
          idxT_hbm.at[:, pl.ds(g * BLK, BLK)], idx_v.at[buf], sems[buf])

    def drain(g, buf):
      pltpu.make_async_copy(
          idxT_hbm.at[:, pl.ds(g * BLK, BLK)], idx_v.at[buf], sems[buf]).wait()

    def block_accum(g, buf):
      def body_j(j, accs):
        out = list(accs)
        for c in range(CB):
          idxv = idx_v[buf, j, pl.ds(c * L, L)]
          out[c] = out[c] + plsc.load_gather(row_v, [idxv])
        return tuple(out)

      accs = lax.fori_loop(
          0, CTX, body_j,
          tuple(jnp.zeros((L,), jnp.float32) for _ in range(CB)),
          unroll=2)
      for c in range(CB):
        pooled_v[pl.ds(g * BLK + c * L, L)] = accs[c] * (1.0 / CTX)

    for p in range(N_PASS):
      d = wid + p * NW
      pltpu.sync_copy(tableT_hbm.at[d], row_v)
      issue(0, 0)

      def body_pair(i, carry):
        g0 = 2 * i
        drain(g0, 0)
        issue(g0 + 1, 1)
        block_accum(g0, 0)
        drain(g0 + 1, 1)

        @pl.when(g0 + 2 < NB)
        def _():
          issue(g0 + 2, 0)

        block_accum(g0 + 1, 1)
        return carry

      lax.fori_loop(0, NB // 2, body_pair, 0)
      pltpu.sync_copy(pooled_v, out_hbm.at[d])

  return sc_kernel(tableT, idxT)


V_TILE = 4096
V_GRID = (VOCAB + V_TILE - 1) // V_TILE


def _mm_body(wt_ref, xt_ref, b_ref, o_ref):
  o_ref[...] = lax.dot_general(
      wt_ref[...], xt_ref[...],
      dimension_numbers=(((0,), (0,)), ((), ())),
      preferred_element_type=jnp.float32) + b_ref[...]


def _tc_project(pooledT, Wt, b2d):
  return pl.pallas_call(
      _mm_body,
      grid=(V_GRID,),
      in_specs=[
          pl.BlockSpec((DIM, V_TILE), lambda i: (0, i)),
          pl.BlockSpec((DIM, BATCH), lambda i: (0, 0)),
          pl.BlockSpec((V_TILE, 1), lambda i: (i, 0)),
      ],
      out_specs=pl.BlockSpec((V_TILE, BATCH), lambda i: (i, 0)),
      out_shape=jax.ShapeDtypeStruct((VOCAB, BATCH), jnp.float32),
  )(Wt, pooledT, b2d)


def kernel(emb_table, W, b, inputs):
  pooledT = _sc_pool_t(emb_table.T, inputs.T)
  out_t = _tc_project(pooledT, W.T, b.reshape(VOCAB, 1))
  return out_t.T


# speedup vs baseline: 1.2573x; 1.1208x over previous
import functools

import jax
import jax.numpy as jnp
from jax import lax
from jax.experimental import pallas as pl
from jax.experimental.pallas import tpu as pltpu
from jax.experimental.pallas import tpu_sc as plsc

VOCAB = 100000
DIM = 64
BATCH = 1024
CTX = 200

NC = 2
NS = 16
NW = NC * NS
L = 16

N_PASS = DIM // NW
BLK = 64
NB = BATCH // BLK
CB = BLK // L


def _sc_pool_t(tableT, idxT):
  @functools.partial(
      pl.kernel,
      out_type=jax.ShapeDtypeStruct((DIM, BATCH), jnp.float32),
      mesh=plsc.VectorSubcoreMesh(core_axis_name="c", subcore_axis_name="s"),
      compiler_params=pltpu.CompilerParams(
          use_tc_tiling_on_sc=False, needs_layout_passes=False),
      scratch_types=[
          pltpu.VMEM((VOCAB,), jnp.float32),
          pltpu.VMEM((2, CTX, BLK), jnp.int32),
          pltpu.VMEM((BATCH,), jnp.float32),
          pltpu.SemaphoreType.DMA,
          pltpu.SemaphoreType.DMA,
      ],
  )
  def sc_kernel(tableT_hbm, idxT_hbm, out_hbm, row_v, idx_v, pooled_v,
                sem0, sem1):
    wid = lax.axis_index("s") * NC + lax.axis_index("c")
    sems = (sem0, sem1)

    def issue(g, buf):
      pltpu.async_copy(
          idxT_hbm.at[:, pl.ds(g * BLK, BLK)], idx_v.at[buf], sems[buf])

    def drain(g, buf):
      pltpu.make_async_copy(
          idxT_hbm.at[:, pl.ds(g * BLK, BLK)], idx_v.at[buf], sems[buf]).wait()

    def block_accum(g, buf):
      def body_j(j, accs):
        out = list(accs)
        for c in range(CB):
          idxv = idx_v[buf, j, pl.ds(c * L, L)]
          out[c] = out[c] + plsc.load_gather(row_v, [idxv])
        return tuple(out)

      accs = lax.fori_loop(
          0, CTX, body_j,
          tuple(jnp.zeros((L,), jnp.float32) for _ in range(CB)),
          unroll=2)
      for c in range(CB):
        pooled_v[pl.ds(g * BLK + c * L, L)] = accs[c] * (1.0 / CTX)

    for p in range(N_PASS):
      d = wid + p * NW
      pltpu.sync_copy(tableT_hbm.at[d], row_v)
      issue(0, 0)

      def body_pair(i, carry):
        g0 = 2 * i
        drain(g0, 0)
        issue(g0 + 1, 1)
        block_accum(g0, 0)
        drain(g0 + 1, 1)

        @pl.when(g0 + 2 < NB)
        def _():
          issue(g0 + 2, 0)

        block_accum(g0 + 1, 1)
        return carry

      lax.fori_loop(0, NB // 2, body_pair, 0)
      pltpu.sync_copy(pooled_v, out_hbm.at[d])

  return sc_kernel(tableT, idxT)


V_TILE = 4096
V_GRID = (VOCAB + V_TILE - 1) // V_TILE


def _mm_body(wt_ref, xt_ref, b_ref, o_ref):
  ones = jnp.ones((1, BATCH), jnp.float32)
  o_ref[...] = lax.dot_general(
      wt_ref[...], xt_ref[...],
      dimension_numbers=(((0,), (0,)), ((), ())),
      preferred_element_type=jnp.float32) + lax.dot_general(
          b_ref[...], ones,
          dimension_numbers=(((0,), (0,)), ((), ())),
          preferred_element_type=jnp.float32)


def _tc_project(pooledT, Wt, b2d):
  return pl.pallas_call(
      _mm_body,
      grid=(V_GRID,),
      in_specs=[
          pl.BlockSpec((DIM, V_TILE), lambda i: (0, i)),
          pl.BlockSpec((DIM, BATCH), lambda i: (0, 0)),
          pl.BlockSpec((1, V_TILE), lambda i: (0, i)),
      ],
      out_specs=pl.BlockSpec((V_TILE, BATCH), lambda i: (i, 0)),
      out_shape=jax.ShapeDtypeStruct((VOCAB, BATCH), jnp.float32),
  )(Wt, pooledT, b2d)


def kernel(emb_table, W, b, inputs):
  pooledT = _sc_pool_t(emb_table.T, inputs.T)
  out_t = _tc_project(pooledT, W.T, b.reshape(1, VOCAB))
  return out_t.T
